# baseline (device time: 38384 ns/iter reference)
import jax
import jax.numpy as jnp
from jax import lax
from jax.experimental import pallas as pl
from jax.experimental.pallas import tpu as pltpu

N_DEV = 4
B, SQ, D = 2, 128, 512
H_LOCAL, DH = 8, 64


def kernel(x, Wq, Wo, Wk, Wv):
    def body(x_ref, wq_ref, wo_ref, wk_ref, wv_ref, out_ref,
             comm_ref, send_sems, recv_sems):
        my_pos = lax.axis_index("i")
        left = (my_pos - 1) % N_DEV
        right = (my_pos + 1) % N_DEV

        barrier_sem = pltpu.get_barrier_semaphore()
        for nbr in [left, right]:
            pl.semaphore_signal(
                barrier_sem, inc=1,
                device_id=(nbr,), device_id_type=pl.DeviceIdType.MESH,
            )
        pl.semaphore_wait(barrier_sem, 2)

        wq = wq_ref[...].astype(jnp.bfloat16)
        wk = wk_ref[...].astype(jnp.bfloat16)
        wv = wv_ref[...].astype(jnp.bfloat16)
        wo = wo_ref[...].astype(jnp.bfloat16)

        for b in range(B):
            xb = x_ref[b].astype(jnp.bfloat16)
            q = jnp.dot(xb, wq, preferred_element_type=jnp.float32)
            k = jnp.dot(xb, wk, preferred_element_type=jnp.float32)
            v = jnp.dot(xb, wv, preferred_element_type=jnp.float32)
            head_outs = []
            for h in range(H_LOCAL):
                sl = slice(h * DH, (h + 1) * DH)
                qh = q[:, sl].astype(jnp.bfloat16)
                kh = k[:, sl].astype(jnp.bfloat16)
                vh = v[:, sl].astype(jnp.bfloat16)
                s = lax.dot_general(
                    qh, kh, (((1,), (1,)), ((), ())),
                    preferred_element_type=jnp.float32,
                ) * 0.125
                m = jnp.max(s, axis=-1, keepdims=True)
                p = jnp.exp(s - m)
                l = jnp.sum(p, axis=-1, keepdims=True)
                head_outs.append(
                    jnp.dot((p / l).astype(jnp.bfloat16), vh,
                            preferred_element_type=jnp.float32)
                )
            attn_b = jnp.concatenate(head_outs, axis=1).astype(jnp.bfloat16)
            partial_b = jnp.dot(attn_b, wo, preferred_element_type=jnp.float32)
            out_ref[b] = partial_b
            comm_ref[0, b] = partial_b

        for h in range(N_DEV - 1):
            rdma = pltpu.make_async_remote_copy(
                src_ref=comm_ref.at[h],
                dst_ref=comm_ref.at[h + 1],
                send_sem=send_sems.at[h],
                recv_sem=recv_sems.at[h],
                device_id=(right,),
                device_id_type=pl.DeviceIdType.MESH,
            )
            rdma.start()
            rdma.wait()
            out_ref[...] += comm_ref[h + 1]

    return pl.pallas_call(
        body,
        out_shape=jax.ShapeDtypeStruct((B, SQ, D), jnp.float32),
        in_specs=[pl.BlockSpec(memory_space=pltpu.VMEM)] * 5,
        out_specs=pl.BlockSpec(memory_space=pltpu.VMEM),
        scratch_shapes=[
            pltpu.VMEM((N_DEV, B, SQ, D), jnp.float32),
            pltpu.SemaphoreType.DMA((N_DEV - 1,)),
            pltpu.SemaphoreType.DMA((N_DEV - 1,)),
        ],
        compiler_params=pltpu.CompilerParams(collective_id=0),
    )(x, Wq, Wo, Wk, Wv)


# device time: 22427 ns/iter; 1.7115x vs baseline; 1.7115x over previous
import jax
import jax.numpy as jnp
from jax import lax
from jax.experimental import pallas as pl
from jax.experimental.pallas import tpu as pltpu

N_DEV = 4
B, SQ, D = 2, 128, 512
H_LOCAL, DH = 8, 64


def kernel(x, Wq, Wo, Wk, Wv):
    def body(x_ref, wq_ref, wo_ref, wk_ref, wv_ref, out_ref,
             send_a, recv_a, send_b, recv_b,
             sa_sems, ra_sems, sb_sems, rb_sems):
        my_pos = lax.axis_index("i")
        partner_a = my_pos ^ 1
        partner_b = 3 - my_pos

        barrier_sem = pltpu.get_barrier_semaphore()
        for nbr in [partner_a, partner_b]:
            pl.semaphore_signal(
                barrier_sem, inc=1,
                device_id=(nbr,), device_id_type=pl.DeviceIdType.MESH,
            )
        pl.semaphore_wait(barrier_sem, 2)

        def exchange(buf_s, buf_r, sem_s, sem_r, b, partner):
            return pltpu.make_async_remote_copy(
                src_ref=buf_s.at[b], dst_ref=buf_r.at[b],
                send_sem=sem_s.at[b], recv_sem=sem_r.at[b],
                device_id=(partner,), device_id_type=pl.DeviceIdType.MESH,
            )

        wq = wq_ref[...].astype(jnp.bfloat16)
        wk = wk_ref[...].astype(jnp.bfloat16)
        wv = wv_ref[...].astype(jnp.bfloat16)
        wo = wo_ref[...].astype(jnp.bfloat16)

        rdma_a = [exchange(send_a, recv_a, sa_sems, ra_sems, b, partner_a)
                  for b in range(B)]
        rdma_b = [exchange(send_b, recv_b, sb_sems, rb_sems, b, partner_b)
                  for b in range(B)]

        for b in range(B):
            xb = x_ref[b].astype(jnp.bfloat16)
            q = jnp.dot(xb, wq, preferred_element_type=jnp.float32)
            k = jnp.dot(xb, wk, preferred_element_type=jnp.float32)
            v = jnp.dot(xb, wv, preferred_element_type=jnp.float32)
            head_outs = []
            for h in range(H_LOCAL):
                sl = slice(h * DH, (h + 1) * DH)
                qh = q[:, sl].astype(jnp.bfloat16)
                kh = k[:, sl].astype(jnp.bfloat16)
                vh = v[:, sl].astype(jnp.bfloat16)
                s = lax.dot_general(
                    qh, kh, (((1,), (1,)), ((), ())),
                    preferred_element_type=jnp.float32,
                ) * 0.125
                m = jnp.max(s, axis=-1, keepdims=True)
                p = jnp.exp(s - m)
                l = jnp.sum(p, axis=-1, keepdims=True)
                head_outs.append(
                    jnp.dot((p / l).astype(jnp.bfloat16), vh,
                            preferred_element_type=jnp.float32)
                )
            attn_b = jnp.concatenate(head_outs, axis=1).astype(jnp.bfloat16)
            partial_b = jnp.dot(attn_b, wo, preferred_element_type=jnp.float32)
            out_ref[b] = partial_b
            send_a[b] = partial_b.astype(jnp.bfloat16)
            rdma_a[b].start()

        for b in range(B):
            rdma_a[b].wait_recv()
            acc = out_ref[b] + recv_a[b].astype(jnp.float32)
            out_ref[b] = acc
            send_b[b] = acc.astype(jnp.bfloat16)
            rdma_b[b].start()

        for b in range(B):
            rdma_b[b].wait_recv()
            out_ref[b] += recv_b[b].astype(jnp.float32)

        for b in range(B):
            rdma_a[b].wait_send()
            rdma_b[b].wait_send()

    return pl.pallas_call(
        body,
        out_shape=jax.ShapeDtypeStruct((B, SQ, D), jnp.float32),
        in_specs=[pl.BlockSpec(memory_space=pltpu.VMEM)] * 5,
        out_specs=pl.BlockSpec(memory_space=pltpu.VMEM),
        scratch_shapes=[
            pltpu.VMEM((B, SQ, D), jnp.bfloat16),
            pltpu.VMEM((B, SQ, D), jnp.bfloat16),
            pltpu.VMEM((B, SQ, D), jnp.bfloat16),
            pltpu.VMEM((B, SQ, D), jnp.bfloat16),
            pltpu.SemaphoreType.DMA((B,)),
            pltpu.SemaphoreType.DMA((B,)),
            pltpu.SemaphoreType.DMA((B,)),
            pltpu.SemaphoreType.DMA((B,)),
        ],
        compiler_params=pltpu.CompilerParams(collective_id=0),
    )(x, Wq, Wo, Wk, Wv)
